# baseline (speedup 1.0000x reference)
import jax
import jax.numpy as jnp
from jax.experimental import pallas as pl
from jax.experimental.pallas import tpu as pltpu

_MIB = 1024 * 1024


def _fused_kernel(x_ref, wd_ref, wm_ref, bd_ref, bm_ref, o_ref, weff_ref):
    t = pl.program_id(1)

    @pl.when(t == 0)
    def _():
        weff_ref[...] = jnp.sum(
            wd_ref[...] * jax.nn.sigmoid(-wm_ref[...]), axis=0
        ).astype(weff_ref.dtype)

    b_eff = jnp.sum(
        bd_ref[...] * jax.nn.sigmoid(-bm_ref[...]), axis=0, keepdims=True)
    xb = x_ref[...].astype(jnp.bfloat16)
    o_ref[...] = jax.lax.dot_general(
        xb, weff_ref[...],
        dimension_numbers=(((1,), (1,)), ((), ())),
        preferred_element_type=jnp.float32) + b_eff


def kernel(x, w_data, w_maskp, b_data, b_maskp):
    B, in_f = x.shape
    ngroup, out_f, _ = w_data.shape

    tn = out_f // 2 if out_f % 256 == 0 else out_f
    tm = 1024 if B % 1024 == 0 else B
    grid = (out_f // tn, B // tm)

    return pl.pallas_call(
        _fused_kernel,
        out_shape=jax.ShapeDtypeStruct((B, out_f), jnp.float32),
        grid=grid,
        in_specs=[
            pl.BlockSpec((tm, in_f), lambda n, t: (t, 0)),
            pl.BlockSpec((ngroup, tn, in_f), lambda n, t: (0, n, 0)),
            pl.BlockSpec((ngroup, tn, in_f), lambda n, t: (0, n, 0)),
            pl.BlockSpec((ngroup, tn), lambda n, t: (0, n)),
            pl.BlockSpec((ngroup, tn), lambda n, t: (0, n)),
        ],
        out_specs=pl.BlockSpec((tm, tn), lambda n, t: (t, n)),
        scratch_shapes=[pltpu.VMEM((tn, in_f), jnp.bfloat16)],
        compiler_params=pltpu.CompilerParams(
            dimension_semantics=("parallel", "arbitrary"),
            vmem_limit_bytes=58 * _MIB),
    )(x, w_data, w_maskp, b_data, b_maskp)


# speedup vs baseline: 1.2491x; 1.2491x over previous
import jax
import jax.numpy as jnp
from jax.experimental import pallas as pl
from jax.experimental.pallas import tpu as pltpu

_MIB = 1024 * 1024


def _collapse_kernel(wd_ref, wm_ref, weff_ref):
    weff_ref[...] = jnp.sum(
        wd_ref[...] * jax.nn.sigmoid(-wm_ref[...]), axis=0
    ).astype(weff_ref.dtype)


def _matmul_kernel(x_ref, w_ref, bd_ref, bm_ref, o_ref):
    b_eff = jnp.sum(
        bd_ref[...] * jax.nn.sigmoid(-bm_ref[...]), axis=0, keepdims=True)
    xb = x_ref[...].astype(jnp.bfloat16)
    o_ref[...] = jax.lax.dot_general(
        xb, w_ref[...],
        dimension_numbers=(((1,), (1,)), ((), ())),
        preferred_element_type=jnp.float32) + b_eff


def kernel(x, w_data, w_maskp, b_data, b_maskp):
    B, in_f = x.shape
    ngroup, out_f, _ = w_data.shape

    tn_c = 256 if out_f % 256 == 0 else out_f
    w_eff = pl.pallas_call(
        _collapse_kernel,
        out_shape=jax.ShapeDtypeStruct((out_f, in_f), jnp.bfloat16),
        grid=(out_f // tn_c,),
        in_specs=[
            pl.BlockSpec((ngroup, tn_c, in_f), lambda j: (0, j, 0)),
            pl.BlockSpec((ngroup, tn_c, in_f), lambda j: (0, j, 0)),
        ],
        out_specs=pl.BlockSpec((tn_c, in_f), lambda j: (j, 0)),
        compiler_params=pltpu.CompilerParams(
            dimension_semantics=("parallel",),
            vmem_limit_bytes=48 * _MIB),
    )(w_data, w_maskp)

    tm = 2048 if B % 2048 == 0 else B
    out = pl.pallas_call(
        _matmul_kernel,
        out_shape=jax.ShapeDtypeStruct((B, out_f), jnp.float32),
        grid=(B // tm,),
        in_specs=[
            pl.BlockSpec((tm, in_f), lambda i: (i, 0)),
            pl.BlockSpec((out_f, in_f), lambda i: (0, 0)),
            pl.BlockSpec((ngroup, out_f), lambda i: (0, 0)),
            pl.BlockSpec((ngroup, out_f), lambda i: (0, 0)),
        ],
        out_specs=pl.BlockSpec((tm, out_f), lambda i: (i, 0)),
        compiler_params=pltpu.CompilerParams(
            dimension_semantics=("parallel",),
            vmem_limit_bytes=48 * _MIB),
    )(x, w_eff, b_data, b_maskp)
    return out


# speedup vs baseline: 1.2544x; 1.0043x over previous
import jax
import jax.numpy as jnp
from jax.experimental import pallas as pl
from jax.experimental.pallas import tpu as pltpu

_MIB = 1024 * 1024


def _collapse_kernel(wd_ref, wm_ref, weff_ref):
    weff_ref[...] = jnp.sum(
        wd_ref[...] * jax.nn.sigmoid(-wm_ref[...]), axis=0
    ).astype(weff_ref.dtype)


def _matmul_kernel(x_ref, w_ref, bd_ref, bm_ref, o_ref):
    b_eff = jnp.sum(
        bd_ref[...] * jax.nn.sigmoid(-bm_ref[...]), axis=0, keepdims=True)
    xb = x_ref[...].astype(jnp.bfloat16)
    o_ref[...] = jax.lax.dot_general(
        xb, w_ref[...],
        dimension_numbers=(((1,), (1,)), ((), ())),
        preferred_element_type=jnp.float32) + b_eff


def kernel(x, w_data, w_maskp, b_data, b_maskp):
    B, in_f = x.shape
    ngroup, out_f, _ = w_data.shape

    tn_c = 256 if out_f % 256 == 0 else out_f
    w_eff = pl.pallas_call(
        _collapse_kernel,
        out_shape=jax.ShapeDtypeStruct((out_f, in_f), jnp.bfloat16),
        grid=(out_f // tn_c,),
        in_specs=[
            pl.BlockSpec((ngroup, tn_c, in_f), lambda j: (0, j, 0)),
            pl.BlockSpec((ngroup, tn_c, in_f), lambda j: (0, j, 0)),
        ],
        out_specs=pl.BlockSpec((tn_c, in_f), lambda j: (j, 0)),
        compiler_params=pltpu.CompilerParams(
            dimension_semantics=("parallel",),
            vmem_limit_bytes=48 * _MIB),
    )(w_data, w_maskp)

    tm = 1024 if B % 1024 == 0 else B
    out = pl.pallas_call(
        _matmul_kernel,
        out_shape=jax.ShapeDtypeStruct((B, out_f), jnp.float32),
        grid=(B // tm,),
        in_specs=[
            pl.BlockSpec((tm, in_f), lambda i: (i, 0)),
            pl.BlockSpec((out_f, in_f), lambda i: (0, 0)),
            pl.BlockSpec((ngroup, out_f), lambda i: (0, 0)),
            pl.BlockSpec((ngroup, out_f), lambda i: (0, 0)),
        ],
        out_specs=pl.BlockSpec((tm, out_f), lambda i: (i, 0)),
        compiler_params=pltpu.CompilerParams(
            dimension_semantics=("parallel",),
            vmem_limit_bytes=48 * _MIB),
    )(x, w_eff, b_data, b_maskp)
    return out
